# baseline (speedup 1.0000x reference)
import jax
import jax.numpy as jnp
from jax import lax
from jax.experimental import pallas as pl
from jax.experimental.pallas import tpu as pltpu

B, N, C = 4, 20000, 80
IMG = 512.0
PRE_NMS_THRESH = 0.05
NMS_THRESH = 0.5
TOP_N = 100

NP = 20480
BL = 2048
SR = 160
SC = 128
NEG = float("-inf")
OFF = 2.0 * IMG


def _stage1(cls_ref, reg_ref, anc_ref,
            s_ref, ox1_ref, oy1_ref, ox2_ref, oy2_ref, cf_ref):
    x = cls_ref[0]
    m = jnp.max(x, axis=0)
    am = jnp.argmax(x, axis=0)
    s = jnp.where(m > PRE_NMS_THRESH, m, NEG)

    r = reg_ref[0]
    a = anc_ref[0]
    a0, a1, a2, a3 = a[0], a[1], a[2], a[3]
    r0, r1, r2, r3 = r[0], r[1], r[2], r[3]
    y_c_a = (a0 + a2) / 2.0
    x_c_a = (a1 + a3) / 2.0
    ha = a2 - a0
    wa = a3 - a1
    w = jnp.exp(r3) * wa
    h = jnp.exp(r2) * ha
    y_c = r0 * ha + y_c_a
    x_c = r1 * wa + x_c_a
    x1 = jnp.clip(x_c - w / 2.0, 0.0, IMG)
    y1 = jnp.clip(y_c - h / 2.0, 0.0, IMG)
    x2 = jnp.clip(x_c + w / 2.0, 0.0, IMG)
    y2 = jnp.clip(y_c + h / 2.0, 0.0, IMG)

    off = am.astype(jnp.float32) * OFF
    s_ref[0, 0] = s
    ox1_ref[0, 0] = x1 + off
    oy1_ref[0, 0] = y1 + off
    ox2_ref[0, 0] = x2 + off
    oy2_ref[0, 0] = y2 + off
    cf_ref[0, 0] = am.astype(jnp.float32)


def _stage2(s_in, ox1_ref, oy1_ref, ox2_ref, oy2_ref, cf_ref,
            o0_ref, o1_ref, o2_ref, o3_ref, s_ref):
    out_refs = [o0_ref, o1_ref, o2_ref, o3_ref]
    s_ref[...] = s_in[...]
    fio = (lax.broadcasted_iota(jnp.int32, (SR, SC), 0) * SC
           + lax.broadcasted_iota(jnp.int32, (SR, SC), 1))
    lane = lax.broadcasted_iota(jnp.int32, (1, SC), 1)

    inv = jnp.where(lane == 5, -1.0, 0.0)
    for r in out_refs:
        r[...] = jnp.broadcast_to(inv, (TOP_N, SC))

    def maxidx(sb):
        m = jnp.max(sb)
        idx = jnp.min(jnp.where(sb == m, fio, jnp.int32(1 << 30)))
        return m, idx

    init = []
    for b in range(B):
        m, i = maxidx(s_ref[pl.ds(b * SR, SR), :])
        done = jnp.logical_not(m > NEG)
        z = jnp.zeros((1, SC), jnp.float32)
        init += [m, i, jnp.int32(0), done, z, z, z, z]

    def cond(carry):
        alldone = carry[3] & carry[11] & carry[19] & carry[27]
        return jnp.logical_not(alldone)

    def body(carry):
        nxt = []
        for b in range(B):
            m, i, cnt, done, ax1, ay1, ax2, ay2 = carry[8 * b:8 * b + 8]
            live = jnp.logical_not(done)
            sl = pl.ds(b * SR, SR)
            eq = fio == i

            def pick(ref):
                return jnp.max(jnp.where(eq, ref[sl, :], NEG))

            wox1 = pick(ox1_ref)
            woy1 = pick(oy1_ref)
            wox2 = pick(ox2_ref)
            woy2 = pick(oy2_ref)
            wcf = pick(cf_ref)

            xx1 = jnp.maximum(wox1, ax1)
            yy1 = jnp.maximum(woy1, ay1)
            xx2 = jnp.minimum(wox2, ax2)
            yy2 = jnp.minimum(woy2, ay2)
            inter = jnp.maximum(xx2 - xx1, 0.0) * jnp.maximum(yy2 - yy1, 0.0)
            a1 = (jnp.maximum(wox2 - wox1, 0.0)
                  * jnp.maximum(woy2 - woy1, 0.0))
            a2 = (jnp.maximum(ax2 - ax1, 0.0)
                  * jnp.maximum(ay2 - ay1, 0.0))
            iou = inter / (a1 + a2 - inter + 1e-8)
            rejected = jnp.max(iou) > NMS_THRESH
            accept = live & jnp.logical_not(rejected)
            selfiou = a1 / (a1 + 1e-8)
            remove = live & (rejected | (selfiou > NMS_THRESH))

            @pl.when(remove)
            def _():
                s_ref[sl, :] = jnp.where(eq, NEG, s_ref[sl, :])

            woff = wcf * OFF
            vals = [wox1 - woff, woy1 - woff, wox2 - woff, woy2 - woff,
                    m, wcf]
            vec = jnp.zeros((1, SC), jnp.float32)
            for q, v in enumerate(vals):
                vec = jnp.where(lane == q, v, vec)

            @pl.when(accept)
            def _():
                out_refs[b][pl.ds(cnt, 1), :] = vec

            amask = (lane == cnt) & accept
            ax1 = jnp.where(amask, wox1, ax1)
            ay1 = jnp.where(amask, woy1, ay1)
            ax2 = jnp.where(amask, wox2, ax2)
            ay2 = jnp.where(amask, woy2, ay2)
            cnt = jnp.where(accept, cnt + 1, cnt)

            m2, i2 = maxidx(s_ref[sl, :])
            m = jnp.where(live & remove, m2, m)
            i = jnp.where(live & remove, i2, i)
            done = done | (cnt >= TOP_N) | jnp.logical_not(m > NEG)
            nxt += [m, i, cnt, done, ax1, ay1, ax2, ay2]
        return tuple(nxt)

    lax.while_loop(cond, body, tuple(init))


@jax.jit
def kernel(imgs, annotations, regression, classification, anchors):
    del imgs, annotations
    cls_p = jnp.pad(classification, ((0, 0), (0, NP - N), (0, 0)),
                    constant_values=-1.0).transpose(0, 2, 1)
    reg_p = jnp.pad(regression, ((0, 0), (0, NP - N), (0, 0))
                    ).transpose(0, 2, 1)
    anc_p = jnp.pad(anchors, ((0, 0), (0, NP - N), (0, 0))
                    ).transpose(0, 2, 1)

    plane = jax.ShapeDtypeStruct((B, 1, NP), jnp.float32)
    planes = pl.pallas_call(
        _stage1,
        grid=(B, NP // BL),
        in_specs=[
            pl.BlockSpec((1, C, BL), lambda b, n: (b, 0, n)),
            pl.BlockSpec((1, 4, BL), lambda b, n: (b, 0, n)),
            pl.BlockSpec((1, 4, BL), lambda b, n: (0, 0, n)),
        ],
        out_specs=[pl.BlockSpec((1, 1, BL), lambda b, n: (b, 0, n))] * 6,
        out_shape=[plane] * 6,
    )(cls_p, reg_p, anc_p)

    slabs = [p.reshape(B * SR, SC) for p in planes]

    o = jax.ShapeDtypeStruct((TOP_N, SC), jnp.float32)
    outs = pl.pallas_call(
        _stage2,
        out_shape=[o] * B,
        scratch_shapes=[pltpu.VMEM((B * SR, SC), jnp.float32)],
    )(*slabs)
    boxes = jnp.stack([ob[:, :4] for ob in outs])
    scores = jnp.stack([ob[:, 4] for ob in outs])
    classes = jnp.stack([ob[:, 5] for ob in outs]).astype(jnp.int32)
    return boxes, scores, classes


# speedup vs baseline: 1.0885x; 1.0885x over previous
import functools

import jax
import jax.numpy as jnp
from jax import lax
from jax.experimental import pallas as pl
from jax.experimental.pallas import tpu as pltpu
from jax.experimental.pallas import tpu_sc as plsc

B, N, C = 4, 20000, 80
IMG = 512.0
PRE_NMS_THRESH = 0.05
NMS_THRESH = 0.5
TOP_N = 100

NP = 20480
BL = 2048
NEG = float("-inf")
OFF = 2.0 * IMG

NT = 16
CH = NP // NT
NCK = CH // 16
NSEG = NCK // 16
L = 16


def _stage1(cls_ref, reg_ref, anc_ref,
            s_ref, ox1_ref, oy1_ref, ox2_ref, oy2_ref, cf_ref):
    x = cls_ref[0]
    m = jnp.max(x, axis=0)
    am = jnp.argmax(x, axis=0)
    s = jnp.where(m > PRE_NMS_THRESH, m, NEG)

    r = reg_ref[0]
    a = anc_ref[0]
    a0, a1, a2, a3 = a[0], a[1], a[2], a[3]
    r0, r1, r2, r3 = r[0], r[1], r[2], r[3]
    y_c_a = (a0 + a2) / 2.0
    x_c_a = (a1 + a3) / 2.0
    ha = a2 - a0
    wa = a3 - a1
    w = jnp.exp(r3) * wa
    h = jnp.exp(r2) * ha
    y_c = r0 * ha + y_c_a
    x_c = r1 * wa + x_c_a
    x1 = jnp.clip(x_c - w / 2.0, 0.0, IMG)
    y1 = jnp.clip(y_c - h / 2.0, 0.0, IMG)
    x2 = jnp.clip(x_c + w / 2.0, 0.0, IMG)
    y2 = jnp.clip(y_c + h / 2.0, 0.0, IMG)

    off = am.astype(jnp.float32) * OFF
    s_ref[0, 0] = s
    ox1_ref[0, 0] = x1 + off
    oy1_ref[0, 0] = y1 + off
    ox2_ref[0, 0] = x2 + off
    oy2_ref[0, 0] = y2 + off
    cf_ref[0, 0] = am.astype(jnp.float32)


def _sc_nms(s_hbm, ox1_hbm, oy1_hbm, ox2_hbm, oy2_hbm, cf_hbm, out_hbm,
            ps, px1, py1, px2, py2, pcf, cmax, ctmp,
            rec_sh, wrec, prec, myslot, candv, acc, outbuf):
    cid = lax.axis_index("c")
    sid = lax.axis_index("s")
    l16 = lax.iota(jnp.int32, L)
    zero16 = jnp.zeros((L,), jnp.float32)
    base = sid * CH
    pools = [ps, px1, py1, px2, py2, pcf]
    hbms = [s_hbm, ox1_hbm, oy1_hbm, ox2_hbm, oy2_hbm, cf_hbm]

    for bi in range(B):
        for pool, hbm in zip(pools, hbms):
            pltpu.sync_copy(hbm.at[bi, pl.ds(base, CH)],
                            pool.at[pl.ds(bi * CH, CH)])

    for bi in range(B):
        def ckbody(k, _):
            ch = ps[pl.ds(bi * CH + k * L, L)]
            ctmp[pl.ds(bi * CH + k * L, L)] = plsc.cummax(ch)
            return 0
        lax.fori_loop(0, NCK, ckbody, 0)
        for sg in range(NSEG):
            idx = bi * CH + (l16 + sg * L) * L + (L - 1)
            cmax[pl.ds(bi * NCK + sg * L, L)] = plsc.load_gather(ctmp, [idx])

    def find_best(bi):
        segs = [cmax[pl.ds(bi * NCK + sg * L, L)] for sg in range(NSEG)]
        g = segs[0]
        for sg in range(1, NSEG):
            g = jnp.maximum(g, segs[sg])
        best = jnp.max(g)
        kc = jnp.int32(1 << 20)
        for sg in range(NSEG):
            cand = jnp.where(segs[sg] == best, l16 + sg * L, jnp.int32(1 << 20))
            kc = jnp.minimum(kc, jnp.min(cand))
        kc = jnp.minimum(kc, jnp.int32(NCK - 1))
        ch = ps[pl.ds(bi * CH + kc * L, L)]
        ln = jnp.min(jnp.where(ch == best, l16, jnp.int32(L)))
        ln = jnp.minimum(ln, jnp.int32(L - 1))
        li = kc * L + ln
        gidx = (base + li).astype(jnp.float32)
        liv = jnp.full((L,), bi * CH + li, jnp.int32)
        vals = [best, gidx,
                plsc.load_gather(px1, [liv]),
                plsc.load_gather(py1, [liv]),
                plsc.load_gather(px2, [liv]),
                plsc.load_gather(py2, [liv]),
                plsc.load_gather(pcf, [liv])]
        vec = zero16
        for q, v in enumerate(vals):
            vec = jnp.where(l16 == q, v, vec)
        myslot[pl.ds(bi * L, L)] = vec
        pltpu.sync_copy(myslot.at[pl.ds(bi * L, L)],
                        rec_sh.at[bi, pl.ds(sid * L, L)])

    for bi in range(B):
        find_best(bi)
        for c4 in range(4):
            for sg in range(8):
                acc[pl.ds((bi * 4 + c4) * 128 + sg * L, L)] = zero16
        @pl.when(sid == 0)
        def _():
            invalid = jnp.where(l16 == 5, -1.0, 0.0)
            def obody(it, _):
                outbuf[pl.ds(bi * TOP_N * L + it * L, L)] = invalid
                return 0
            lax.fori_loop(0, TOP_N, obody, 0)
    plsc.subcore_barrier()

    sv, iv = [], []
    for bi in range(B):
        pltpu.sync_copy(rec_sh.at[bi], candv.at[pl.ds(bi * NT * L, NT * L)])
        sv.append(plsc.load_gather(candv, [bi * NT * L + l16 * L]))
        iv.append(plsc.load_gather(candv, [bi * NT * L + l16 * L + 1]))
    plsc.subcore_barrier()

    def cond(carry):
        d = carry[4 * B]
        for bi in range(1, B):
            d = d & carry[4 * B + bi]
        return jnp.logical_not(d)

    def body(carry):
        s16 = list(carry[0:B])
        i16 = list(carry[B:2 * B])
        wtp = list(carry[2 * B:3 * B])
        cnt = list(carry[3 * B:4 * B])
        done = list(carry[4 * B:5 * B])
        winners = []
        for bi in range(B):
            pltpu.sync_copy(rec_sh.at[bi, pl.ds(wtp[bi] * L, L)],
                            prec.at[pl.ds(bi * L, L)])
            pv = prec[pl.ds(bi * L, L)]
            ns = pv[0]
            ni = pv[1]
            s16[bi] = jnp.where(l16 == wtp[bi], ns, s16[bi])
            i16[bi] = jnp.where(l16 == wtp[bi], ni, i16[bi])

            best = jnp.max(s16[bi])
            gidx = jnp.min(jnp.where(s16[bi] == best, i16[bi],
                                     jnp.float32(1 << 24)))
            wt = jnp.int32(gidx) // CH
            live = jnp.logical_not(done[bi]) & (best > NEG)
            pltpu.sync_copy(rec_sh.at[bi, pl.ds(wt * L, L)],
                            wrec.at[pl.ds(bi * L, L)])
            wv = wrec[pl.ds(bi * L, L)]
            wox1 = wv[2]
            woy1 = wv[3]
            wox2 = wv[4]
            woy2 = wv[5]
            wcf = wv[6]

            a1 = (jnp.maximum(wox2 - wox1, 0.0)
                  * jnp.maximum(woy2 - woy1, 0.0))
            riou = zero16
            for sg in range(8):
                ax1 = acc[pl.ds((bi * 4 + 0) * 128 + sg * L, L)]
                ay1 = acc[pl.ds((bi * 4 + 1) * 128 + sg * L, L)]
                ax2 = acc[pl.ds((bi * 4 + 2) * 128 + sg * L, L)]
                ay2 = acc[pl.ds((bi * 4 + 3) * 128 + sg * L, L)]
                xx1 = jnp.maximum(wox1, ax1)
                yy1 = jnp.maximum(woy1, ay1)
                xx2 = jnp.minimum(wox2, ax2)
                yy2 = jnp.minimum(woy2, ay2)
                inter = (jnp.maximum(xx2 - xx1, 0.0)
                         * jnp.maximum(yy2 - yy1, 0.0))
                a2 = (jnp.maximum(ax2 - ax1, 0.0)
                      * jnp.maximum(ay2 - ay1, 0.0))
                riou = jnp.maximum(riou, inter / (a1 + a2 - inter + 1e-8))
            rejected = jnp.max(riou) > NMS_THRESH
            accept = live & jnp.logical_not(rejected)
            av = jnp.full((L,), a1)
            selfiou = (av / (av + av - av + 1e-8))[0]
            remove = live & (rejected | (selfiou > NMS_THRESH))

            seg = cnt[bi] // L
            lnc = cnt[bi] % L
            wvals = [wox1, woy1, wox2, woy2]
            @pl.when(accept)
            def _():
                for c4 in range(4):
                    o = (bi * 4 + c4) * 128 + seg * L
                    avv = acc[pl.ds(o, L)]
                    acc[pl.ds(o, L)] = jnp.where(l16 == lnc, wvals[c4], avv)
            @pl.when(accept & (sid == 0))
            def _():
                woffs = wcf * OFF
                ovals = [wox1 - woffs, woy1 - woffs, wox2 - woffs,
                         woy2 - woffs, best, wcf]
                ovec = zero16
                for q, v in enumerate(ovals):
                    ovec = jnp.where(l16 == q, v, ovec)
                outbuf[pl.ds(bi * TOP_N * L + cnt[bi] * L, L)] = ovec

            cnt[bi] = jnp.where(accept, cnt[bi] + 1, cnt[bi])
            done[bi] = (done[bi] | (cnt[bi] >= TOP_N)
                        | jnp.logical_not(best > NEG))
            winners.append((wt, remove))
            wtp[bi] = wt

        plsc.subcore_barrier()

        for bi in range(B):
            wt, remove = winners[bi]
            @pl.when(remove & (sid == wt))
            def _():
                li = jnp.int32(jnp.min(jnp.where(
                    s16[bi] == jnp.max(s16[bi]), i16[bi],
                    jnp.float32(1 << 24)))) - base
                kc = li // L
                ln = li % L
                ch = ps[pl.ds(bi * CH + kc * L, L)]
                ch = jnp.where(l16 == ln, NEG, ch)
                ps[pl.ds(bi * CH + kc * L, L)] = ch
                cm = jnp.max(ch)
                sg2 = kc // L
                lo = kc % L
                o = bi * NCK + sg2 * L
                seg_v = cmax[pl.ds(o, L)]
                cmax[pl.ds(o, L)] = jnp.where(l16 == lo, cm, seg_v)
                find_best(bi)

        plsc.subcore_barrier()
        return tuple(s16 + i16 + wtp + cnt + done)

    init = (sv + iv + [jnp.int32(0)] * B + [jnp.int32(0)] * B
            + [jnp.logical_not(jnp.max(sv[bi]) > NEG) for bi in range(B)])
    lax.while_loop(cond, body, tuple(init))

    @pl.when((sid == 0) & (cid == 0))
    def _():
        pltpu.sync_copy(outbuf, out_hbm)


@jax.jit
def kernel(imgs, annotations, regression, classification, anchors):
    del imgs, annotations
    cls_p = jnp.pad(classification, ((0, 0), (0, NP - N), (0, 0)),
                    constant_values=-1.0).transpose(0, 2, 1)
    reg_p = jnp.pad(regression, ((0, 0), (0, NP - N), (0, 0))
                    ).transpose(0, 2, 1)
    anc_p = jnp.pad(anchors, ((0, 0), (0, NP - N), (0, 0))
                    ).transpose(0, 2, 1)

    plane = jax.ShapeDtypeStruct((B, 1, NP), jnp.float32)
    planes = pl.pallas_call(
        _stage1,
        grid=(B, NP // BL),
        in_specs=[
            pl.BlockSpec((1, C, BL), lambda b, n: (b, 0, n)),
            pl.BlockSpec((1, 4, BL), lambda b, n: (b, 0, n)),
            pl.BlockSpec((1, 4, BL), lambda b, n: (0, 0, n)),
        ],
        out_specs=[pl.BlockSpec((1, 1, BL), lambda b, n: (b, 0, n))] * 6,
        out_shape=[plane] * 6,
    )(cls_p, reg_p, anc_p)

    flats = [p.reshape(B, NP) for p in planes]

    mesh = plsc.VectorSubcoreMesh(core_axis_name="c", subcore_axis_name="s")
    out = pl.kernel(
        _sc_nms,
        out_type=jax.ShapeDtypeStruct((B * TOP_N * L,), jnp.float32),
        mesh=mesh,
        compiler_params=pltpu.CompilerParams(needs_layout_passes=False),
        scratch_types=(
            [pltpu.VMEM((B * CH,), jnp.float32)] * 6
            + [pltpu.VMEM((B * NCK,), jnp.float32)]
            + [pltpu.VMEM((B * CH,), jnp.float32)]
            + [pltpu.VMEM_SHARED((B, NT * L), jnp.float32)]
            + [pltpu.VMEM((B * L,), jnp.float32)] * 3
            + [pltpu.VMEM((B * NT * L,), jnp.float32)]
            + [pltpu.VMEM((B * 4 * 128,), jnp.float32)]
            + [pltpu.VMEM((B * TOP_N * L,), jnp.float32)]
        ),
    )(*flats)

    out = out.reshape(B, TOP_N, L)
    boxes = out[:, :, 0:4]
    scores = out[:, :, 4]
    classes = out[:, :, 5].astype(jnp.int32)
    return boxes, scores, classes


# speedup vs baseline: 1.3735x; 1.2618x over previous
import functools

import jax
import jax.numpy as jnp
from jax import lax
from jax.experimental import pallas as pl
from jax.experimental.pallas import tpu as pltpu
from jax.experimental.pallas import tpu_sc as plsc

B, N, C = 4, 20000, 80
IMG = 512.0
PRE_NMS_THRESH = 0.05
NMS_THRESH = 0.5
TOP_N = 100

NP = 20480
BL = 2048
NEG = float("-inf")
OFF = 2.0 * IMG

NT = 16
CH = NP // NT
NCK = CH // 16
NSEG = NCK // 16
L = 16


def _stage1(cls_ref, reg_ref, anc_ref,
            s_ref, ox1_ref, oy1_ref, ox2_ref, oy2_ref, cf_ref):
    n = pl.program_id(1)
    x = cls_ref[0]
    m = jnp.max(x, axis=0)
    am = jnp.argmax(x, axis=0)
    absn = n * BL + lax.broadcasted_iota(jnp.int32, (BL,), 0)
    s = jnp.where((absn < N) & (m > PRE_NMS_THRESH), m, NEG)

    r = reg_ref[0]
    a = anc_ref[0]
    a0, a1, a2, a3 = a[0], a[1], a[2], a[3]
    r0, r1, r2, r3 = r[0], r[1], r[2], r[3]
    y_c_a = (a0 + a2) / 2.0
    x_c_a = (a1 + a3) / 2.0
    ha = a2 - a0
    wa = a3 - a1
    w = jnp.exp(r3) * wa
    h = jnp.exp(r2) * ha
    y_c = r0 * ha + y_c_a
    x_c = r1 * wa + x_c_a
    x1 = jnp.clip(x_c - w / 2.0, 0.0, IMG)
    y1 = jnp.clip(y_c - h / 2.0, 0.0, IMG)
    x2 = jnp.clip(x_c + w / 2.0, 0.0, IMG)
    y2 = jnp.clip(y_c + h / 2.0, 0.0, IMG)

    off = am.astype(jnp.float32) * OFF
    s_ref[0, 0] = s
    ox1_ref[0, 0] = x1 + off
    oy1_ref[0, 0] = y1 + off
    ox2_ref[0, 0] = x2 + off
    oy2_ref[0, 0] = y2 + off
    cf_ref[0, 0] = am.astype(jnp.float32)


def _sc_nms(s_hbm, ox1_hbm, oy1_hbm, ox2_hbm, oy2_hbm, cf_hbm, out_hbm,
            ps, px1, py1, px2, py2, pcf, cmax, ctmp,
            rec_sh, prec, myslot, candv, acc, outbuf, sem):
    cid = lax.axis_index("c")
    sid = lax.axis_index("s")
    l16 = lax.iota(jnp.int32, L)
    zero16 = jnp.zeros((L,), jnp.float32)
    base = sid * CH
    pools = [ps, px1, py1, px2, py2, pcf]
    hbms = [s_hbm, ox1_hbm, oy1_hbm, ox2_hbm, oy2_hbm, cf_hbm]

    for bi in range(B):
        for pool, hbm in zip(pools, hbms):
            pltpu.sync_copy(hbm.at[bi, pl.ds(base, CH)],
                            pool.at[pl.ds(bi * CH, CH)])

    for bi in range(B):
        def ckbody(k, _):
            ch = ps[pl.ds(bi * CH + k * L, L)]
            ctmp[pl.ds(bi * CH + k * L, L)] = plsc.cummax(ch)
            return 0
        lax.fori_loop(0, NCK, ckbody, 0)
        for sg in range(NSEG):
            idx = bi * CH + (l16 + sg * L) * L + (L - 1)
            cmax[pl.ds(bi * NCK + sg * L, L)] = plsc.load_gather(ctmp, [idx])

    def find_best(bi, par):
        segs = [cmax[pl.ds(bi * NCK + sg * L, L)] for sg in range(NSEG)]
        g = segs[0]
        for sg in range(1, NSEG):
            g = jnp.maximum(g, segs[sg])
        best = jnp.max(g)
        kc = jnp.int32(1 << 20)
        for sg in range(NSEG):
            cand = jnp.where(segs[sg] == best, l16 + sg * L, jnp.int32(1 << 20))
            kc = jnp.minimum(kc, jnp.min(cand))
        kc = jnp.minimum(kc, jnp.int32(NCK - 1))
        ch = ps[pl.ds(bi * CH + kc * L, L)]
        ln = jnp.min(jnp.where(ch == best, l16, jnp.int32(L)))
        ln = jnp.minimum(ln, jnp.int32(L - 1))
        li = kc * L + ln
        gidx = (base + li).astype(jnp.float32)
        liv = jnp.full((L,), bi * CH + li, jnp.int32)
        vals = [best, gidx,
                plsc.load_gather(px1, [liv]),
                plsc.load_gather(py1, [liv]),
                plsc.load_gather(px2, [liv]),
                plsc.load_gather(py2, [liv]),
                plsc.load_gather(pcf, [liv])]
        vec = zero16
        for q, v in enumerate(vals):
            vec = jnp.where(l16 == q, v, vec)
        myslot[pl.ds(bi * L, L)] = vec
        pltpu.sync_copy(myslot.at[pl.ds(bi * L, L)],
                        rec_sh.at[par, bi, pl.ds(sid * L, L)])

    for bi in range(B):
        find_best(bi, 0)
        for c4 in range(4):
            for sg in range(8):
                acc[pl.ds((bi * 4 + c4) * 128 + sg * L, L)] = zero16
        @pl.when(sid == 0)
        def _():
            invalid = jnp.where(l16 == 5, -1.0, 0.0)
            def obody(it, _):
                outbuf[pl.ds(bi * TOP_N * L + it * L, L)] = invalid
                return 0
            lax.fori_loop(0, TOP_N, obody, 0)
    plsc.subcore_barrier()

    vecs = []
    for bi in range(B):
        pltpu.sync_copy(rec_sh.at[0, bi], candv.at[pl.ds(bi * NT * L, NT * L)])
        vecs.append([plsc.load_gather(candv, [bi * NT * L + l16 * L + f])
                     for f in range(7)])
    plsc.subcore_barrier()

    def substep(par, state):
        vs, wtp, cnt, done = state
        nxt = 1 - par
        waits = []
        for bi in range(B):
            waits.append(pltpu.async_copy(
                rec_sh.at[par, bi, pl.ds(wtp[bi] * L, L)],
                prec.at[pl.ds(bi * L, L)], sem))
        for w in waits:
            w.wait()
        winners = []
        for bi in range(B):
            pv = prec[pl.ds(bi * L, L)]
            nv = list(vs[bi])
            for f in range(7):
                nv[f] = jnp.where(l16 == wtp[bi], pv[f], nv[f])
            vs[bi] = nv
            s16, i16, x1v, y1v, x2v, y2v, cfv = nv

            best = jnp.max(s16)
            gidx = jnp.min(jnp.where(s16 == best, i16, jnp.float32(1 << 24)))
            wt = jnp.minimum(jnp.maximum(jnp.int32(gidx) // CH, 0),
                             jnp.int32(NT - 1))
            live = jnp.logical_not(done[bi]) & (best > NEG)
            sel = l16 == wt
            wox1 = jnp.max(jnp.where(sel, x1v, NEG))
            woy1 = jnp.max(jnp.where(sel, y1v, NEG))
            wox2 = jnp.max(jnp.where(sel, x2v, NEG))
            woy2 = jnp.max(jnp.where(sel, y2v, NEG))
            wcf = jnp.max(jnp.where(sel, cfv, NEG))

            a1 = (jnp.maximum(wox2 - wox1, 0.0)
                  * jnp.maximum(woy2 - woy1, 0.0))
            nsegs = (cnt[bi] + (L - 1)) // L
            def ioubody(sg, riou):
                o = (bi * 4) * 128 + sg * L
                ax1 = acc[pl.ds(o, L)]
                ay1 = acc[pl.ds(o + 128, L)]
                ax2 = acc[pl.ds(o + 256, L)]
                ay2 = acc[pl.ds(o + 384, L)]
                xx1 = jnp.maximum(wox1, ax1)
                yy1 = jnp.maximum(woy1, ay1)
                xx2 = jnp.minimum(wox2, ax2)
                yy2 = jnp.minimum(woy2, ay2)
                inter = (jnp.maximum(xx2 - xx1, 0.0)
                         * jnp.maximum(yy2 - yy1, 0.0))
                a2 = (jnp.maximum(ax2 - ax1, 0.0)
                      * jnp.maximum(ay2 - ay1, 0.0))
                return jnp.maximum(riou, inter / (a1 + a2 - inter + 1e-8))
            riou = lax.fori_loop(0, nsegs, ioubody, zero16)
            rejected = jnp.max(riou) > NMS_THRESH
            accept = live & jnp.logical_not(rejected)
            av = jnp.full((L,), a1)
            selfiou = (av / (av + av - av + 1e-8))[0]
            remove = live & (rejected | (selfiou > NMS_THRESH))

            seg = cnt[bi] // L
            lnc = cnt[bi] % L
            wvals = [wox1, woy1, wox2, woy2]
            @pl.when(accept)
            def _():
                for c4 in range(4):
                    o = (bi * 4 + c4) * 128 + seg * L
                    avv = acc[pl.ds(o, L)]
                    acc[pl.ds(o, L)] = jnp.where(l16 == lnc, wvals[c4], avv)
            @pl.when(accept & (sid == 0))
            def _():
                woffs = wcf * OFF
                ovals = [wox1 - woffs, woy1 - woffs, wox2 - woffs,
                         woy2 - woffs, best, wcf]
                ovec = zero16
                for q, v in enumerate(ovals):
                    ovec = jnp.where(l16 == q, v, ovec)
                outbuf[pl.ds(bi * TOP_N * L + cnt[bi] * L, L)] = ovec

            cnt[bi] = jnp.where(accept, cnt[bi] + 1, cnt[bi])
            done[bi] = (done[bi] | (cnt[bi] >= TOP_N)
                        | jnp.logical_not(best > NEG))
            winners.append((wt, remove, live))
            wtp[bi] = wt

        for bi in range(B):
            wt, remove, live = winners[bi]
            @pl.when(remove & (sid == wt))
            def _():
                s16, i16 = vs[bi][0], vs[bi][1]
                li = jnp.int32(jnp.min(jnp.where(
                    s16 == jnp.max(s16), i16, jnp.float32(1 << 24)))) - base
                kc = li // L
                ln = li % L
                ch = ps[pl.ds(bi * CH + kc * L, L)]
                ch = jnp.where(l16 == ln, NEG, ch)
                ps[pl.ds(bi * CH + kc * L, L)] = ch
                cm = jnp.max(ch)
                sg2 = kc // L
                lo = kc % L
                o = bi * NCK + sg2 * L
                seg_v = cmax[pl.ds(o, L)]
                cmax[pl.ds(o, L)] = jnp.where(l16 == lo, cm, seg_v)
                find_best(bi, nxt)
            @pl.when(live & jnp.logical_not(remove) & (sid == wt))
            def _():
                pltpu.sync_copy(myslot.at[pl.ds(bi * L, L)],
                                rec_sh.at[nxt, bi, pl.ds(sid * L, L)])

        plsc.subcore_barrier()
        return vs, wtp, cnt, done

    def cond(carry):
        d = carry[28 + 2 * B]
        for bi in range(1, B):
            d = d & carry[28 + 2 * B + bi]
        return jnp.logical_not(d)

    def body(carry):
        vs = [list(carry[7 * bi:7 * bi + 7]) for bi in range(B)]
        wtp = list(carry[28:28 + B])
        cnt = list(carry[28 + B:28 + 2 * B])
        done = list(carry[28 + 2 * B:28 + 3 * B])
        state = (vs, wtp, cnt, done)
        state = substep(0, state)
        state = substep(1, state)
        vs, wtp, cnt, done = state
        flat = []
        for bi in range(B):
            flat += vs[bi]
        return tuple(flat + wtp + cnt + done)

    flat0 = []
    for bi in range(B):
        flat0 += vecs[bi]
    init = tuple(flat0 + [jnp.int32(0)] * B + [jnp.int32(0)] * B
                 + [jnp.logical_not(jnp.max(vecs[bi][0]) > NEG)
                    for bi in range(B)])
    lax.while_loop(cond, body, init)

    @pl.when((sid == 0) & (cid == 0))
    def _():
        pltpu.sync_copy(outbuf, out_hbm)


@jax.jit
def kernel(imgs, annotations, regression, classification, anchors):
    del imgs, annotations
    cls_p = classification.transpose(0, 2, 1)
    reg_p = jnp.pad(regression, ((0, 0), (0, NP - N), (0, 0))
                    ).transpose(0, 2, 1)
    anc_p = jnp.pad(anchors, ((0, 0), (0, NP - N), (0, 0))
                    ).transpose(0, 2, 1)

    plane = jax.ShapeDtypeStruct((B, 1, NP), jnp.float32)
    planes = pl.pallas_call(
        _stage1,
        grid=(B, NP // BL),
        in_specs=[
            pl.BlockSpec((1, C, BL), lambda b, n: (b, 0, n)),
            pl.BlockSpec((1, 4, BL), lambda b, n: (b, 0, n)),
            pl.BlockSpec((1, 4, BL), lambda b, n: (0, 0, n)),
        ],
        out_specs=[pl.BlockSpec((1, 1, BL), lambda b, n: (b, 0, n))] * 6,
        out_shape=[plane] * 6,
    )(cls_p, reg_p, anc_p)

    flats = [p.reshape(B, NP) for p in planes]

    mesh = plsc.VectorSubcoreMesh(core_axis_name="c", subcore_axis_name="s")
    out = pl.kernel(
        _sc_nms,
        out_type=jax.ShapeDtypeStruct((B * TOP_N * L,), jnp.float32),
        mesh=mesh,
        compiler_params=pltpu.CompilerParams(needs_layout_passes=False),
        scratch_types=(
            [pltpu.VMEM((B * CH,), jnp.float32)] * 6
            + [pltpu.VMEM((B * NCK,), jnp.float32)]
            + [pltpu.VMEM((B * CH,), jnp.float32)]
            + [pltpu.VMEM_SHARED((2, B, NT * L), jnp.float32)]
            + [pltpu.VMEM((B * L,), jnp.float32)] * 2
            + [pltpu.VMEM((B * NT * L,), jnp.float32)]
            + [pltpu.VMEM((B * 4 * 128,), jnp.float32)]
            + [pltpu.VMEM((B * TOP_N * L,), jnp.float32)]
            + [pltpu.SemaphoreType.DMA]
        ),
    )(*flats)

    out = out.reshape(B, TOP_N, L)
    boxes = out[:, :, 0:4]
    scores = out[:, :, 4]
    classes = out[:, :, 5].astype(jnp.int32)
    return boxes, scores, classes
